# Initial kernel scaffold; baseline (speedup 1.0000x reference)
import jax
import jax.numpy as jnp
from jax.experimental import pallas as pl


def kernel(input_tensor, cells_to_chans):
    raise NotImplementedError("write your pallas kernel here")


# speedup vs baseline: 1.9282x; 1.9282x over previous
import functools

import jax
import jax.numpy as jnp
from jax import lax
from jax.experimental import pallas as pl
from jax.experimental.pallas import tpu as pltpu
from jax.experimental.pallas import tpu_sc as plsc

F_IN = 16
R = 512
C = 4096
NCELLS = 4096
W = 8
NB = (F_IN * R) // W
NWORKERS = 32
BRICKS_PER_WORKER = NB // NWORKERS
GROUPS = NCELLS // 16


def _sc_body(xb_hbm, c8_hbm, out_hbm, x_s, t01_s, c8_s):
  wid = lax.axis_index("s") * 2 + lax.axis_index("c")
  pltpu.sync_copy(c8_hbm, c8_s)

  iota = lax.iota(jnp.int32, 16)
  k1 = iota & 7
  k2 = (iota >> 3) << 3
  halfmask = iota < 8
  zero16 = jnp.zeros((16,), jnp.float32)

  def brick_body(t, carry):
    b = wid * BRICKS_PER_WORKER + t
    pltpu.sync_copy(xb_hbm.at[b], x_s)

    def zbody(i, c):
      for k in range(8):
        t01_s[pl.ds(i * 128 + k * 16, 16)] = zero16
      return c

    lax.fori_loop(0, C * 16 // 128, zbody, 0)

    def gbody(g, c):
      base = g * 16
      for t16 in range(16):
        cvt = []
        for j in range(3):
          bvec = jnp.broadcast_to(base + (j * NCELLS + t16), (16,)).astype(
              jnp.int32)
          cvt.append(plsc.load_gather(c8_s, [bvec]))
        u = [cvt[j] + k1 for j in range(3)]
        a = [plsc.load_gather(x_s, [u[j]]) for j in range(3)]
        q0 = a[1] * a[2]
        q1 = a[0] * a[2]
        q2 = a[0] * a[1]
        mp3 = q0 * a[0]
        for j, qj in ((0, q0), (1, q1), (2, q2)):
          s = u[j] + cvt[j] + k2
          w = jnp.where(halfmask, mp3, qj)
          plsc.addupdate_scatter(t01_s, [s], w)
      return c

    lax.fori_loop(0, GROUPS, gbody, 0)
    pltpu.sync_copy(t01_s, out_hbm.at[b])
    return carry

  lax.fori_loop(0, BRICKS_PER_WORKER, brick_body, 0)


@jax.jit
def kernel(input_tensor, cells_to_chans):
  f_in, r, c = input_tensor.shape
  xb = input_tensor.reshape(NB, W, c).transpose(0, 2, 1).reshape(NB, c * W)
  c8 = (cells_to_chans.astype(jnp.int32) * W).reshape(-1)

  mesh = plsc.VectorSubcoreMesh(core_axis_name="c", subcore_axis_name="s")
  out = pl.kernel(
      _sc_body,
      out_type=jax.ShapeDtypeStruct((NB, c * 16), jnp.float32),
      mesh=mesh,
      scratch_types=[
          pltpu.VMEM((c * W,), jnp.float32),
          pltpu.VMEM((c * 16,), jnp.float32),
          pltpu.VMEM((3 * NCELLS,), jnp.int32),
      ],
      compiler_params=pltpu.CompilerParams(needs_layout_passes=False),
  )(xb, c8)

  t = out.reshape(NB, c, 2, W).transpose(2, 0, 3, 1)
  return t.reshape(2 * f_in, r, c)
